# baseline (speedup 1.0000x reference)
import functools

import jax
import jax.numpy as jnp
from jax.experimental import pallas as pl


def _rowsq(v):
    v = v * v
    while v.shape[-1] > 1:
        h = v.shape[-1] // 2
        v = v[:, :h] + v[:, h:]
    return v[:, 0]


def _sims_body(x_blk_ref, x_ref, out_ref):
    xi = x_blk_ref[...]
    xa = x_ref[...]
    aa = _rowsq(xi)
    bb = _rowsq(xa)
    mm = jax.lax.dot_general(xi, xa, (((1,), (1,)), ((), ())))
    sq = aa[:, None] + bb[None, :] - 2.0 * mm
    sq = jnp.maximum(sq, 1e-12)
    out_ref[...] = jnp.exp(-jnp.sqrt(sq) / x_ref.shape[-1])


def _sims_matrix(x):
    n, d = x.shape
    br = 256
    return pl.pallas_call(
        _sims_body,
        grid=(n // br,),
        in_specs=[
            pl.BlockSpec((br, d), lambda i: (i, 0)),
            pl.BlockSpec((n, d), lambda i: (0, 0)),
        ],
        out_specs=pl.BlockSpec((br, n), lambda i: (i, 0)),
        out_shape=jax.ShapeDtypeStruct((n, n), jnp.float32),
    )(x, x)


def kernel(x):
    n = x.shape[0]
    sims = _sims_matrix(x)
    k = n * n
    flat = sims.reshape(-1)
    vals, idx = jax.lax.top_k(flat, k)
    row = idx // n
    col = idx % n
    edge_index = jnp.stack([row, col], axis=0)
    return edge_index, vals
